# baseline (speedup 1.0000x reference)
import functools

import jax
import jax.numpy as jnp
import numpy as np
from jax import lax
from jax.experimental import pallas as pl
from jax.experimental.pallas import tpu as pltpu
from jax.experimental.pallas import tpu_sc as plsc

B = 16384
D = 128

NC = 2
NS = 16
NW = NC * NS
LANES = 16

CHUNK = 128
NF32 = 4
NBF = 4

SROWS = 2 * B
ROWS_PER_W = SROWS // NW
N_CHUNKS = ROWS_PER_W // CHUNK

_PB = np.stack([np.arange(LANES), np.arange(LANES) + LANES], axis=1).reshape(-1)
_PERM_D = np.concatenate([_PB + 32 * q for q in range(D // 32)])
PERM_2D = np.concatenate([_PERM_D, _PERM_D + D])


def _sc_gather_bf16(table, idx_grp):
    mesh = plsc.VectorSubcoreMesh(core_axis_name="c", subcore_axis_name="s")

    @functools.partial(
        pl.kernel,
        mesh=mesh,
        out_type=jax.ShapeDtypeStruct((SROWS, D), jnp.bfloat16),
        scratch_types=[
            pltpu.VMEM((N_CHUNKS, CHUNK), jnp.int32),
            pltpu.VMEM((NF32, CHUNK, D), jnp.float32),
            pltpu.VMEM((NBF, CHUNK // 2, D), jnp.uint32),
        ]
        + [pltpu.SemaphoreType.DMA] * (NF32 + NBF),
    )
    def gather_kernel(table_hbm, idx_hbm, out_hbm, idx_v, rows_f, rows_w,
                      *sems):
        gsems, wsems = sems[:NF32], sems[NF32:]
        wid = lax.axis_index("s") * NC + lax.axis_index("c")
        base = wid * ROWS_PER_W
        pltpu.sync_copy(idx_hbm.at[wid], idx_v)
        gcopy = [None] * N_CHUNKS
        wcopy = [None] * N_CHUNKS
        for c in range(NF32):
            gcopy[c] = pltpu.async_copy(
                table_hbm.at[idx_v.at[c]], rows_f.at[c], gsems[c])

        for c in range(N_CHUNKS):
            bf = c % NF32
            bb = c % NBF
            gcopy[c].wait()
            if c >= NBF:
                wcopy[c - NBF].wait()

            def cvt_pair(r2, _, bf=bf, bb=bb):
                half = jnp.uint32(0x8000)
                himask = jnp.uint32(0xFFFF0000)
                u = 2 * r2
                v = u + 1
                for t in range(D // LANES):
                    a = lax.bitcast_convert_type(
                        rows_f[bf, u, pl.ds(LANES * t, LANES)], jnp.uint32)
                    bvec = lax.bitcast_convert_type(
                        rows_f[bf, v, pl.ds(LANES * t, LANES)], jnp.uint32)
                    word = ((a + half) >> 16) | ((bvec + half) & himask)
                    rows_w[bb, r2, pl.ds(LANES * t, LANES)] = word
                return 0

            lax.fori_loop(0, CHUNK // 2, cvt_pair, 0, unroll=2)

            wcopy[c] = pltpu.async_copy(
                rows_w.at[bb],
                out_hbm.bitcast(jnp.uint32).at[
                    pl.ds(pl.multiple_of((base + c * CHUNK) // 2, 64),
                          CHUNK // 2)],
                wsems[bb])
            nxt = c + NF32
            if nxt < N_CHUNKS:
                gcopy[nxt] = pltpu.async_copy(
                    table_hbm.at[idx_v.at[nxt]], rows_f.at[bf], gsems[bf])
        for c in range(max(0, N_CHUNKS - NBF), N_CHUNKS):
            wcopy[c].wait()

    return gather_kernel(table, idx_grp)


BLK = 8192


def _tc_linear(g, Wp, bias):
    def body(g0_ref, g1_ref, w_ref, b_ref, o_ref):
        dn = (((1,), (1,)), ((), ()))
        acc = lax.dot_general(g0_ref[...], w_ref[:, :D], dn,
                              preferred_element_type=jnp.float32)
        acc = acc + lax.dot_general(g1_ref[...], w_ref[:, D:], dn,
                                    preferred_element_type=jnp.float32)
        o_ref[...] = acc + b_ref[...]

    nblk = B // BLK
    return pl.pallas_call(
        body,
        grid=(nblk,),
        in_specs=[
            pl.BlockSpec((BLK, D), lambda i: (i, 0)),
            pl.BlockSpec((BLK, D), lambda i: (i + nblk, 0)),
            pl.BlockSpec((D, 2 * D), lambda i: (0, 0)),
            pl.BlockSpec((1, D), lambda i: (0, 0)),
        ],
        out_specs=pl.BlockSpec((BLK, D), lambda i: (i, 0)),
        out_shape=jax.ShapeDtypeStruct((B, D), jnp.float32),
    )(g, g, Wp, bias)


def kernel(x, table, W, b):
    idx_grp = x.astype(jnp.int32).T.reshape(NW, N_CHUNKS, CHUNK)
    g = _sc_gather_bf16(table, idx_grp)
    Wp = W.astype(jnp.bfloat16)
    return _tc_linear(g, Wp, b.reshape(1, D))


# speedup vs baseline: 1.4259x; 1.4259x over previous
import functools

import jax
import jax.numpy as jnp
from jax import lax
from jax.experimental import pallas as pl
from jax.experimental.pallas import tpu as pltpu
from jax.experimental.pallas import tpu_sc as plsc

B = 16384
D = 128

NC = 2
NS = 16
NW = NC * NS

CHUNK = 128
MAX_NBUF = 7

NSLICE = 1
BS = B // NSLICE
SROWS = 2 * BS
ROWS_PER_W = SROWS // NW
N_CHUNKS = ROWS_PER_W // CHUNK
NBUF = min(MAX_NBUF, N_CHUNKS)


def _sc_gather(table, idx_grp):
    mesh = plsc.VectorSubcoreMesh(core_axis_name="c", subcore_axis_name="s")

    @functools.partial(
        pl.kernel,
        mesh=mesh,
        out_type=jax.ShapeDtypeStruct((SROWS, D), jnp.float32),
        scratch_types=[
            pltpu.VMEM((N_CHUNKS, CHUNK), jnp.int32),
            pltpu.VMEM((NBUF, CHUNK, D), jnp.float32),
        ]
        + [pltpu.SemaphoreType.DMA] * (2 * NBUF),
    )
    def gather_kernel(table_hbm, idx_hbm, out_hbm, idx_v, rows_v, *sems):
        gsems, wsems = sems[:NBUF], sems[NBUF:]
        wid = lax.axis_index("s") * NC + lax.axis_index("c")
        base = wid * ROWS_PER_W
        gcopy = [None] * N_CHUNKS
        wcopy = [None] * N_CHUNKS
        pltpu.sync_copy(idx_hbm.at[wid], idx_v)
        for c in range(NBUF):
            gcopy[c] = pltpu.async_copy(
                table_hbm.at[idx_v.at[c]], rows_v.at[c], gsems[c])
        for c in range(N_CHUNKS):
            buf = c % NBUF
            gcopy[c].wait()
            wcopy[c] = pltpu.async_copy(
                rows_v.at[buf],
                out_hbm.at[pl.ds(base + c * CHUNK, CHUNK)],
                wsems[buf])
            nxt = c + NBUF
            if nxt < N_CHUNKS:
                wcopy[c].wait()
                gcopy[nxt] = pltpu.async_copy(
                    table_hbm.at[idx_v.at[nxt]], rows_v.at[buf], gsems[buf])
        for c in range(max(0, N_CHUNKS - NBUF), N_CHUNKS):
            wcopy[c].wait()

    return gather_kernel(table, idx_grp)


BLK = 8192


def _tc_linear(g, W, bias):
    def body(g0_ref, g1_ref, w_ref, b_ref, o_ref):
        dn = (((1,), (1,)), ((), ()))
        acc = lax.dot_general(g0_ref[...], w_ref[:, :D], dn,
                              preferred_element_type=jnp.float32)
        acc = acc + lax.dot_general(g1_ref[...], w_ref[:, D:], dn,
                                    preferred_element_type=jnp.float32)
        o_ref[...] = acc + b_ref[...]

    nblk = BS // BLK
    return pl.pallas_call(
        body,
        grid=(nblk,),
        in_specs=[
            pl.BlockSpec((BLK, D), lambda i: (i, 0)),
            pl.BlockSpec((BLK, D), lambda i: (i + nblk, 0)),
            pl.BlockSpec((D, 2 * D), lambda i: (0, 0)),
            pl.BlockSpec((1, D), lambda i: (0, 0)),
        ],
        out_specs=pl.BlockSpec((BLK, D), lambda i: (i, 0)),
        out_shape=jax.ShapeDtypeStruct((BS, D), jnp.float32),
    )(g, g, W, bias)


def kernel(x, table, W, b):
    xi = x.astype(jnp.int32)
    bias = b.reshape(1, D)
    outs = []
    for s in range(NSLICE):
        idx_grp = (xi[s * BS:(s + 1) * BS]
                   .T.reshape(NW, N_CHUNKS, CHUNK))
        g = _sc_gather(table, idx_grp)
        outs.append(_tc_linear(g, W, bias))
    if NSLICE == 1:
        return outs[0]
    return jnp.concatenate(outs, axis=0)
